# baseline (speedup 1.0000x reference)
import functools
import math

import numpy as np
import jax
import jax.numpy as jnp
from jax import lax
from jax.experimental import pallas as pl
from jax.experimental.pallas import tpu as pltpu
from jax.experimental.pallas import tpu_sc as plsc

C_N = 384
C_Z = 128
C_HID = 16
H = 12
PQ = 4
PV = 8
N = 4096
K = 32
INF = 1e8
EPS = 1e-6

D_KV = 2 * H * C_HID
D_PTS = 3 * H * (PQ + PV)
D_TAB = D_KV + D_PTS
D_TABP = 896
D_QA = 576
NKP = H * PQ
NVP = H * PV

NB = 64
R = NB * K
N_BLOCKS = N // NB

QK_SCALE = math.sqrt(1.0 / (3 * C_HID))
B_SCALE = math.sqrt(1.0 / 3)
PT_SCALE = math.sqrt(1.0 / (3 * (PQ * 9.0 / 2)))

HIGH = jax.lax.Precision.HIGHEST


def _kv_col_perm():
    src = np.zeros(D_KV, np.int32)
    for h in range(H):
        for c in range(C_HID):
            src[h * C_HID + c] = h * 2 * C_HID + c
            src[H * C_HID + h * C_HID + c] = h * 2 * C_HID + C_HID + c
    return src


def _kv_pts_col_perm():
    P = PQ + PV
    src = np.zeros(D_PTS, np.int32)
    for j in range(3):
        for h in range(H):
            for p in range(P):
                s_col = j * H * P + h * P + p
                if p < PQ:
                    t = j * H * P + h * PQ + p
                else:
                    t = j * H * P + H * PQ + h * PV + (p - PQ)
                src[t] = s_col
    return src


_KV_PERM = _kv_col_perm()
_KV_PTS_PERM = _kv_pts_col_perm()


def _project_body(s_ref, wqa_ref, wtab_ref, qa_ref, tab_ref):
    s = s_ref[...]
    qa_ref[...] = jnp.dot(s, wqa_ref[...],
                          preferred_element_type=jnp.float32)
    tab_ref[...] = jnp.dot(s, wtab_ref[...],
                           preferred_element_type=jnp.float32)


def _project(s2, w_qa, w_tab):
    blk = 512
    return pl.pallas_call(
        _project_body,
        grid=(N // blk,),
        in_specs=[
            pl.BlockSpec((blk, C_N), lambda i: (i, 0)),
            pl.BlockSpec((C_N, D_QA), lambda i: (0, 0)),
            pl.BlockSpec((C_N, D_TABP), lambda i: (0, 0)),
        ],
        out_specs=[
            pl.BlockSpec((blk, D_QA), lambda i: (i, 0)),
            pl.BlockSpec((blk, D_TABP), lambda i: (i, 0)),
        ],
        out_shape=[
            jax.ShapeDtypeStruct((N, D_QA), jnp.float32),
            jax.ShapeDtypeStruct((N, D_TABP), jnp.float32),
        ],
        compiler_params=pltpu.CompilerParams(
            dimension_semantics=("arbitrary",)),
    )(s2, w_qa, w_tab)


_GW = 64
_NW = 32


def _sc_gather(table, idx1d, n_rows):
    bpw = n_rows // _NW
    mesh = plsc.VectorSubcoreMesh(core_axis_name="c", subcore_axis_name="s")

    @functools.partial(
        pl.kernel,
        out_type=jax.ShapeDtypeStruct((n_rows, D_TABP), jnp.float32),
        mesh=mesh,
        scratch_types=[
            pltpu.VMEM((bpw,), jnp.int32),
            pltpu.VMEM((_GW, D_TABP), jnp.float32),
            pltpu.SemaphoreType.DMA,
        ],
    )
    def gk(table_hbm, idx_hbm, out_hbm, idx_v, buf, sem):
        wid = lax.axis_index("s") * 2 + lax.axis_index("c")
        base = wid * bpw
        pltpu.sync_copy(idx_hbm.at[pl.ds(base, bpw)], idx_v)

        @pl.loop(0, bpw // _GW)
        def _(i):
            pltpu.async_copy(
                table_hbm.at[idx_v.at[pl.ds(i * _GW, _GW)]], buf, sem).wait()
            pltpu.sync_copy(buf, out_hbm.at[pl.ds(base + i * _GW, _GW)])

    return gk(table, idx1d)


def _attn_body(qa_ref, ge_ref, ze_ref, mp_ref, mq_ref, ms_ref, mt_ref,
               pm_ref, wb_ref, hw_ref, wout_ref, bout_ref, out_ref):
    f32 = jnp.float32

    def qbc(lo, w):
        q = qa_ref[:, lo:lo + w]
        return jnp.broadcast_to(q[:, None, :], (NB, K, w)).reshape(R, w)

    q3 = qbc(0, 192)
    qp0 = qbc(256, NKP)
    qp1 = qbc(384, NKP)
    qp2 = qbc(512, NKP)

    ze = ze_ref[...]
    bz = jnp.dot(ze, wb_ref[...],
                 preferred_element_type=f32)

    prod = q3 * ge_ref[:, 0:192]
    seg192 = (lax.broadcasted_iota(jnp.int32, (192, H), 0) // C_HID
              == lax.broadcasted_iota(jnp.int32, (192, H), 1)).astype(f32)
    aqk = jnp.dot(prod, seg192,
                  preferred_element_type=f32)

    X = ge_ref[:, 384:528]
    Y = ge_ref[:, 528:672]
    Z = ge_ref[:, 672:816]

    m9 = mp_ref[...] * mq_ref[...] - ms_ref[...] * mt_ref[...]

    def mcol(i):
        return m9[:, i:i + 1]

    c0 = mcol(0) * X + mcol(1) * Y + mcol(2) * Z
    c1 = mcol(3) * X + mcol(4) * Y + mcol(5) * Z
    c2 = mcol(6) * X + mcol(7) * Y + mcol(8) * Z

    sq = ((qp0 - c0[:, :NKP]) ** 2
          + (qp1 - c1[:, :NKP]) ** 2
          + (qp2 - c2[:, :NKP]) ** 2)
    seg48 = (lax.broadcasted_iota(jnp.int32, (NKP, H), 0) // PQ
             == lax.broadcasted_iota(jnp.int32, (NKP, H), 1)).astype(f32)
    ptat = jnp.dot(sq, seg48,
                   preferred_element_type=f32)
    hw = jnp.log(1.0 + jnp.exp(hw_ref[0:1, :H])) * PT_SCALE

    logits = (aqk * QK_SCALE + bz * B_SCALE - 0.5 * (ptat * hw)
              + INF * (pm_ref[:, 0:1] - 1.0))

    l3 = logits.reshape(NB, K, H)
    m = jnp.max(l3, axis=1, keepdims=True)
    e3 = jnp.exp(l3 - m)
    ssum = jnp.sum(e3, axis=1, keepdims=True)
    a = (e3 / ssum).reshape(R, H)

    exp192 = (lax.broadcasted_iota(jnp.int32, (H, 192), 1) // C_HID
              == lax.broadcasted_iota(jnp.int32, (H, 192), 0)).astype(f32)
    a192 = jnp.dot(a, exp192, preferred_element_type=f32)
    exp96 = (lax.broadcasted_iota(jnp.int32, (H, NVP), 1) // PV
             == lax.broadcasted_iota(jnp.int32, (H, NVP), 0)).astype(f32)
    a96 = jnp.dot(a, exp96, preferred_element_type=f32)

    smat = (lax.broadcasted_iota(jnp.int32, (NB, R), 1) // K
            == lax.broadcasted_iota(jnp.int32, (NB, R), 0)).astype(f32)
    o = jnp.dot(smat, a192 * ge_ref[:, 192:384], preferred_element_type=f32)
    op0 = jnp.dot(smat, a96 * c0[:, NKP:], preferred_element_type=f32)
    op1 = jnp.dot(smat, a96 * c1[:, NKP:], preferred_element_type=f32)
    op2 = jnp.dot(smat, a96 * c2[:, NKP:], preferred_element_type=f32)
    onorm = jnp.sqrt(op0 * op0 + op1 * op1 + op2 * op2 + EPS)

    acc = jnp.dot(o, wout_ref[0:192, :], preferred_element_type=f32)
    acc += jnp.dot(op0, wout_ref[192:288, :], preferred_element_type=f32)
    acc += jnp.dot(op1, wout_ref[288:384, :], preferred_element_type=f32)
    acc += jnp.dot(op2, wout_ref[384:480, :], preferred_element_type=f32)
    acc += jnp.dot(onorm, wout_ref[480:576, :], preferred_element_type=f32)

    for h in range(H):
        oph = jnp.dot(smat, a[:, h:h + 1] * ze, preferred_element_type=f32)
        acc += jnp.dot(oph, wout_ref[576 + h * C_Z:576 + (h + 1) * C_Z, :],
                       preferred_element_type=f32)

    out_ref[...] = acc + bout_ref[0:1, :]


def _attn(qa, ge, ze, mp, mq, ms, mt, pm, w_b, hw_pad, w_out, bout_pad,
          n_nodes):
    cod = H * (C_Z + C_HID + PV * 4)
    return pl.pallas_call(
        _attn_body,
        grid=(n_nodes // NB,),
        in_specs=[
            pl.BlockSpec((NB, D_QA), lambda i: (i, 0)),
            pl.BlockSpec((R, D_TABP), lambda i: (i, 0)),
            pl.BlockSpec((R, C_Z), lambda i: (i, 0)),
            pl.BlockSpec((R, 9), lambda i: (i, 0)),
            pl.BlockSpec((R, 9), lambda i: (i, 0)),
            pl.BlockSpec((R, 9), lambda i: (i, 0)),
            pl.BlockSpec((R, 9), lambda i: (i, 0)),
            pl.BlockSpec((R, 1), lambda i: (i, 0)),
            pl.BlockSpec((C_Z, H), lambda i: (0, 0)),
            pl.BlockSpec((8, 16), lambda i: (0, 0)),
            pl.BlockSpec((cod, C_N), lambda i: (0, 0)),
            pl.BlockSpec((8, C_N), lambda i: (0, 0)),
        ],
        out_specs=pl.BlockSpec((NB, C_N), lambda i: (i, 0)),
        out_shape=jax.ShapeDtypeStruct((n_nodes, C_N), jnp.float32),
        compiler_params=pltpu.CompilerParams(
            dimension_semantics=("arbitrary",)),
    )(qa, ge, ze, mp, mq, ms, mt, pm, w_b, hw_pad, w_out, bout_pad)


def kernel(s, z_e, r, direction, rel_ori, pair_mask, E_idx,
           w_q, w_kv, w_q_pts, w_kv_pts, w_b, head_weights, w_out, b_out):
    del r
    s2 = s.reshape(N, C_N)
    w_qa = jnp.zeros((C_N, D_QA), jnp.float32)
    w_qa = w_qa.at[:, 0:192].set(w_q)
    w_qa = w_qa.at[:, 256:304].set(w_q_pts[:, 0:48])
    w_qa = w_qa.at[:, 384:432].set(w_q_pts[:, 48:96])
    w_qa = w_qa.at[:, 512:560].set(w_q_pts[:, 96:144])
    w_tab = jnp.concatenate(
        [w_kv[:, _KV_PERM], w_kv_pts[:, _KV_PTS_PERM],
         jnp.zeros((C_N, D_TABP - D_TAB), jnp.float32)], axis=1)

    qa, table = _project(s2, w_qa, w_tab)

    idx1d = E_idx.astype(jnp.int32).reshape(N * K)
    ze = z_e.reshape(N * K, C_Z)
    ro9 = rel_ori.reshape(N * K, 9)
    d3 = direction.reshape(N * K, 3)
    ii = np.arange(9) // 3
    jj = np.arange(9) % 3
    mp = d3[:, (ii + 1) % 3]
    ms = d3[:, (ii + 2) % 3]
    mq = ro9[:, ((ii + 2) % 3) * 3 + jj]
    mt = ro9[:, ((ii + 1) % 3) * 3 + jj]
    pm = pair_mask.reshape(N * K, 1)
    hw_pad = jnp.zeros((8, 16), jnp.float32).at[0, :H].set(head_weights)
    bout_pad = jnp.zeros((8, C_N), jnp.float32).at[0].set(b_out)

    ge = _sc_gather(table, idx1d, N * K)
    out = _attn(qa, ge, ze, mp, mq, ms, mt, pm, w_b, hw_pad, w_out,
                bout_pad, N)
    return out.reshape(1, N, C_N)


# speedup vs baseline: 1.0318x; 1.0318x over previous
import functools
import math

import numpy as np
import jax
import jax.numpy as jnp
from jax import lax
from jax.experimental import pallas as pl
from jax.experimental.pallas import tpu as pltpu
from jax.experimental.pallas import tpu_sc as plsc

C_N = 384
C_Z = 128
C_HID = 16
H = 12
PQ = 4
PV = 8
N = 4096
K = 32
INF = 1e8
EPS = 1e-6

D_KV = 2 * H * C_HID
D_PTS = 3 * H * (PQ + PV)
D_TAB = D_KV + D_PTS
D_TABP = 896
D_QA = 576
NKP = H * PQ
NVP = H * PV

NB = 64
R = NB * K
N_BLOCKS = N // NB

QK_SCALE = math.sqrt(1.0 / (3 * C_HID))
B_SCALE = math.sqrt(1.0 / 3)
PT_SCALE = math.sqrt(1.0 / (3 * (PQ * 9.0 / 2)))

HIGH = jax.lax.Precision.HIGHEST


def _kv_col_perm():
    src = np.zeros(D_KV, np.int32)
    for h in range(H):
        for c in range(C_HID):
            src[h * C_HID + c] = h * 2 * C_HID + c
            src[H * C_HID + h * C_HID + c] = h * 2 * C_HID + C_HID + c
    return src


def _kv_pts_col_perm():
    P = PQ + PV
    src = np.zeros(D_PTS, np.int32)
    for j in range(3):
        for h in range(H):
            for p in range(P):
                s_col = j * H * P + h * P + p
                if p < PQ:
                    t = j * H * P + h * PQ + p
                else:
                    t = j * H * P + H * PQ + h * PV + (p - PQ)
                src[t] = s_col
    return src


_KV_PERM = _kv_col_perm()
_KV_PTS_PERM = _kv_pts_col_perm()


def _project_body(s_ref, wqa_ref, wtab_ref, qa_ref, tab_ref):
    s = s_ref[...]
    qa_ref[...] = jnp.dot(s, wqa_ref[...],
                          preferred_element_type=jnp.float32)
    tab_ref[...] = jnp.dot(s, wtab_ref[...],
                           preferred_element_type=jnp.float32)


def _project(s2, w_qa, w_tab):
    blk = 512
    return pl.pallas_call(
        _project_body,
        grid=(N // blk,),
        in_specs=[
            pl.BlockSpec((blk, C_N), lambda i: (i, 0)),
            pl.BlockSpec((C_N, D_QA), lambda i: (0, 0)),
            pl.BlockSpec((C_N, D_TABP), lambda i: (0, 0)),
        ],
        out_specs=[
            pl.BlockSpec((blk, D_QA), lambda i: (i, 0)),
            pl.BlockSpec((blk, D_TABP), lambda i: (i, 0)),
        ],
        out_shape=[
            jax.ShapeDtypeStruct((N, D_QA), jnp.float32),
            jax.ShapeDtypeStruct((N, D_TABP), jnp.float32),
        ],
        compiler_params=pltpu.CompilerParams(
            dimension_semantics=("arbitrary",)),
    )(s2, w_qa, w_tab)


_GW = 64
_NW = 32


def _sc_gather(table, idx1d, n_rows):
    bpw = n_rows // _NW
    mesh = plsc.VectorSubcoreMesh(core_axis_name="c", subcore_axis_name="s")

    @functools.partial(
        pl.kernel,
        out_type=jax.ShapeDtypeStruct((n_rows, D_TABP), jnp.float32),
        mesh=mesh,
        scratch_types=[
            pltpu.VMEM((bpw,), jnp.int32),
            pltpu.VMEM((_GW, D_TABP), jnp.float32),
            pltpu.SemaphoreType.DMA,
        ],
    )
    def gk(table_hbm, idx_hbm, out_hbm, idx_v, buf, sem):
        wid = lax.axis_index("s") * 2 + lax.axis_index("c")
        base = wid * bpw
        pltpu.sync_copy(idx_hbm.at[pl.ds(base, bpw)], idx_v)

        @pl.loop(0, bpw // _GW)
        def _(i):
            pltpu.async_copy(
                table_hbm.at[idx_v.at[pl.ds(i * _GW, _GW)]], buf, sem).wait()
            pltpu.sync_copy(buf, out_hbm.at[pl.ds(base + i * _GW, _GW)])

    return gk(table, idx1d)


def _attn_body(qa_ref, ge_ref, ze_ref, mp_ref, mq_ref, ms_ref, mt_ref,
               pm_ref, wb_ref, hw_ref, wout_ref, bout_ref, out_ref):
    f32 = jnp.float32

    def qbc(lo, w):
        q = qa_ref[:, lo:lo + w]
        return jnp.broadcast_to(q[:, None, :], (NB, K, w)).reshape(R, w)

    q3 = qbc(0, 192)
    qp0 = qbc(256, NKP)
    qp1 = qbc(384, NKP)
    qp2 = qbc(512, NKP)

    ze = ze_ref[...]
    bz = jnp.dot(ze, wb_ref[...],
                 preferred_element_type=f32)

    prod = q3 * ge_ref[:, 0:192]
    seg192 = (lax.broadcasted_iota(jnp.int32, (192, H), 0) // C_HID
              == lax.broadcasted_iota(jnp.int32, (192, H), 1)).astype(f32)
    aqk = jnp.dot(prod, seg192,
                  preferred_element_type=f32)

    X = ge_ref[:, 384:528]
    Y = ge_ref[:, 528:672]
    Z = ge_ref[:, 672:816]

    m9 = mp_ref[...] * mq_ref[...] - ms_ref[...] * mt_ref[...]

    def mcol(i):
        return m9[:, i:i + 1]

    c0 = mcol(0) * X + mcol(1) * Y + mcol(2) * Z
    c1 = mcol(3) * X + mcol(4) * Y + mcol(5) * Z
    c2 = mcol(6) * X + mcol(7) * Y + mcol(8) * Z

    sq = ((qp0 - c0[:, :NKP]) ** 2
          + (qp1 - c1[:, :NKP]) ** 2
          + (qp2 - c2[:, :NKP]) ** 2)
    seg48 = (lax.broadcasted_iota(jnp.int32, (NKP, H), 0) // PQ
             == lax.broadcasted_iota(jnp.int32, (NKP, H), 1)).astype(f32)
    ptat = jnp.dot(sq, seg48,
                   preferred_element_type=f32)
    hw = jnp.log(1.0 + jnp.exp(hw_ref[0:1, :H])) * PT_SCALE

    logits = (aqk * QK_SCALE + bz * B_SCALE - 0.5 * (ptat * hw)
              + INF * (pm_ref[:, 0:1] - 1.0))

    l3 = logits.reshape(NB, K, H)
    m = jnp.max(l3, axis=1, keepdims=True)
    e3 = jnp.exp(l3 - m)
    ssum = jnp.sum(e3, axis=1, keepdims=True)
    a = (e3 / ssum).reshape(R, H)

    exp192 = (lax.broadcasted_iota(jnp.int32, (H, 192), 1) // C_HID
              == lax.broadcasted_iota(jnp.int32, (H, 192), 0)).astype(f32)
    a192 = jnp.dot(a, exp192, preferred_element_type=f32)
    exp96 = (lax.broadcasted_iota(jnp.int32, (H, NVP), 1) // PV
             == lax.broadcasted_iota(jnp.int32, (H, NVP), 0)).astype(f32)
    a96 = jnp.dot(a, exp96, preferred_element_type=f32)

    o = (a192 * ge_ref[:, 192:384]).reshape(NB, K, 192).sum(axis=1)
    op0 = (a96 * c0[:, NKP:]).reshape(NB, K, NVP).sum(axis=1)
    op1 = (a96 * c1[:, NKP:]).reshape(NB, K, NVP).sum(axis=1)
    op2 = (a96 * c2[:, NKP:]).reshape(NB, K, NVP).sum(axis=1)
    onorm = jnp.sqrt(op0 * op0 + op1 * op1 + op2 * op2 + EPS)

    acc = jnp.dot(o, wout_ref[0:192, :], preferred_element_type=f32)
    acc += jnp.dot(op0, wout_ref[192:288, :], preferred_element_type=f32)
    acc += jnp.dot(op1, wout_ref[288:384, :], preferred_element_type=f32)
    acc += jnp.dot(op2, wout_ref[384:480, :], preferred_element_type=f32)
    acc += jnp.dot(onorm, wout_ref[480:576, :], preferred_element_type=f32)

    for h in range(H):
        oph = (a[:, h:h + 1] * ze).reshape(NB, K, C_Z).sum(axis=1)
        acc += jnp.dot(oph, wout_ref[576 + h * C_Z:576 + (h + 1) * C_Z, :],
                       preferred_element_type=f32)

    out_ref[...] = acc + bout_ref[0:1, :]


def _attn(qa, ge, ze, mp, mq, ms, mt, pm, w_b, hw_pad, w_out, bout_pad,
          n_nodes):
    cod = H * (C_Z + C_HID + PV * 4)
    return pl.pallas_call(
        _attn_body,
        grid=(n_nodes // NB,),
        in_specs=[
            pl.BlockSpec((NB, D_QA), lambda i: (i, 0)),
            pl.BlockSpec((R, D_TABP), lambda i: (i, 0)),
            pl.BlockSpec((R, C_Z), lambda i: (i, 0)),
            pl.BlockSpec((R, 9), lambda i: (i, 0)),
            pl.BlockSpec((R, 9), lambda i: (i, 0)),
            pl.BlockSpec((R, 9), lambda i: (i, 0)),
            pl.BlockSpec((R, 9), lambda i: (i, 0)),
            pl.BlockSpec((R, 1), lambda i: (i, 0)),
            pl.BlockSpec((C_Z, H), lambda i: (0, 0)),
            pl.BlockSpec((8, 16), lambda i: (0, 0)),
            pl.BlockSpec((cod, C_N), lambda i: (0, 0)),
            pl.BlockSpec((8, C_N), lambda i: (0, 0)),
        ],
        out_specs=pl.BlockSpec((NB, C_N), lambda i: (i, 0)),
        out_shape=jax.ShapeDtypeStruct((n_nodes, C_N), jnp.float32),
        compiler_params=pltpu.CompilerParams(
            dimension_semantics=("arbitrary",)),
    )(qa, ge, ze, mp, mq, ms, mt, pm, w_b, hw_pad, w_out, bout_pad)


def kernel(s, z_e, r, direction, rel_ori, pair_mask, E_idx,
           w_q, w_kv, w_q_pts, w_kv_pts, w_b, head_weights, w_out, b_out):
    del r
    s2 = s.reshape(N, C_N)
    w_qa = jnp.zeros((C_N, D_QA), jnp.float32)
    w_qa = w_qa.at[:, 0:192].set(w_q)
    w_qa = w_qa.at[:, 256:304].set(w_q_pts[:, 0:48])
    w_qa = w_qa.at[:, 384:432].set(w_q_pts[:, 48:96])
    w_qa = w_qa.at[:, 512:560].set(w_q_pts[:, 96:144])
    w_tab = jnp.concatenate(
        [w_kv[:, _KV_PERM], w_kv_pts[:, _KV_PTS_PERM],
         jnp.zeros((C_N, D_TABP - D_TAB), jnp.float32)], axis=1)

    qa, table = _project(s2, w_qa, w_tab)

    idx1d = E_idx.astype(jnp.int32).reshape(N * K)
    ze = z_e.reshape(N * K, C_Z)
    ro9 = rel_ori.reshape(N * K, 9)
    d3 = direction.reshape(N * K, 3)
    ii = np.arange(9) // 3
    jj = np.arange(9) % 3
    mp = d3[:, (ii + 1) % 3]
    ms = d3[:, (ii + 2) % 3]
    mq = ro9[:, ((ii + 2) % 3) * 3 + jj]
    mt = ro9[:, ((ii + 1) % 3) * 3 + jj]
    pm = pair_mask.reshape(N * K, 1)
    hw_pad = jnp.zeros((8, 16), jnp.float32).at[0, :H].set(head_weights)
    bout_pad = jnp.zeros((8, C_N), jnp.float32).at[0].set(b_out)

    ge = _sc_gather(table, idx1d, N * K)
    out = _attn(qa, ge, ze, mp, mq, ms, mt, pm, w_b, hw_pad, w_out,
                bout_pad, N)
    return out.reshape(1, N, C_N)


# speedup vs baseline: 1.0806x; 1.0474x over previous
import functools
import math

import numpy as np
import jax
import jax.numpy as jnp
from jax import lax
from jax.experimental import pallas as pl
from jax.experimental.pallas import tpu as pltpu
from jax.experimental.pallas import tpu_sc as plsc

C_N = 384
C_Z = 128
C_HID = 16
H = 12
PQ = 4
PV = 8
N = 4096
K = 32
INF = 1e8
EPS = 1e-6

D_KV = 2 * H * C_HID
D_PTS = 3 * H * (PQ + PV)
D_TAB = D_KV + D_PTS
D_TABP = 896
D_QA = 576
NKP = H * PQ
NVP = H * PV

NB = 64
R = NB * K
N_BLOCKS = N // NB

QK_SCALE = math.sqrt(1.0 / (3 * C_HID))
B_SCALE = math.sqrt(1.0 / 3)
PT_SCALE = math.sqrt(1.0 / (3 * (PQ * 9.0 / 2)))

HIGH = jax.lax.Precision.HIGHEST


def _kv_col_perm():
    src = np.zeros(D_KV, np.int32)
    for h in range(H):
        for c in range(C_HID):
            src[h * C_HID + c] = h * 2 * C_HID + c
            src[H * C_HID + h * C_HID + c] = h * 2 * C_HID + C_HID + c
    return src


def _kv_pts_col_perm():
    P = PQ + PV
    src = np.zeros(D_PTS, np.int32)
    for j in range(3):
        for h in range(H):
            for p in range(P):
                s_col = j * H * P + h * P + p
                if p < PQ:
                    t = j * H * P + h * PQ + p
                else:
                    t = j * H * P + H * PQ + h * PV + (p - PQ)
                src[t] = s_col
    return src


_KV_PERM = _kv_col_perm()
_KV_PTS_PERM = _kv_pts_col_perm()


def _project_body(s_ref, wqa_ref, wtab_ref, qa_ref, tab_ref):
    s = s_ref[...]
    qa_ref[...] = jnp.dot(s, wqa_ref[...],
                          preferred_element_type=jnp.float32)
    tab_ref[...] = jnp.dot(s, wtab_ref[...],
                           preferred_element_type=jnp.float32)


def _project(s2, w_qa, w_tab):
    blk = 512
    return pl.pallas_call(
        _project_body,
        grid=(N // blk,),
        in_specs=[
            pl.BlockSpec((blk, C_N), lambda i: (i, 0)),
            pl.BlockSpec((C_N, D_QA), lambda i: (0, 0)),
            pl.BlockSpec((C_N, D_TABP), lambda i: (0, 0)),
        ],
        out_specs=[
            pl.BlockSpec((blk, D_QA), lambda i: (i, 0)),
            pl.BlockSpec((blk, D_TABP), lambda i: (i, 0)),
        ],
        out_shape=[
            jax.ShapeDtypeStruct((N, D_QA), jnp.float32),
            jax.ShapeDtypeStruct((N, D_TABP), jnp.float32),
        ],
        compiler_params=pltpu.CompilerParams(
            dimension_semantics=("arbitrary",)),
    )(s2, w_qa, w_tab)


_GW = 64
_NW = 32


def _sc_gather(table, idx1d, n_rows):
    bpw = n_rows // _NW
    mesh = plsc.VectorSubcoreMesh(core_axis_name="c", subcore_axis_name="s")

    @functools.partial(
        pl.kernel,
        out_type=jax.ShapeDtypeStruct((n_rows, D_TABP), jnp.float32),
        mesh=mesh,
        scratch_types=[
            pltpu.VMEM((bpw,), jnp.int32),
            pltpu.VMEM((_GW, D_TABP), jnp.float32),
            pltpu.VMEM((_GW, D_TABP), jnp.float32),
            pltpu.SemaphoreType.DMA,
            pltpu.SemaphoreType.DMA,
        ],
    )
    def gk(table_hbm, idx_hbm, out_hbm, idx_v, buf0, buf1, sem0, sem1):
        wid = lax.axis_index("s") * 2 + lax.axis_index("c")
        base = wid * bpw
        pltpu.sync_copy(idx_hbm.at[pl.ds(base, bpw)], idx_v)
        nch = bpw // _GW

        def start(i, buf, sem):
            pltpu.async_copy(
                table_hbm.at[idx_v.at[pl.ds(i * _GW, _GW)]], buf, sem)

        def finish(i, buf, sem):
            pltpu.make_async_copy(
                table_hbm.at[idx_v.at[pl.ds(i * _GW, _GW)]], buf, sem).wait()
            pltpu.sync_copy(buf, out_hbm.at[pl.ds(base + i * _GW, _GW)])

        start(0, buf0, sem0)

        @pl.loop(0, nch // 2 - 1)
        def _(it):
            i0 = it * 2
            start(i0 + 1, buf1, sem1)
            finish(i0, buf0, sem0)
            start(i0 + 2, buf0, sem0)
            finish(i0 + 1, buf1, sem1)

        start(nch - 1, buf1, sem1)
        finish(nch - 2, buf0, sem0)
        finish(nch - 1, buf1, sem1)

    return gk(table, idx1d)


def _attn_body(qa_ref, ge_ref, ze_ref, ro_ref, dr_ref, pm_ref,
               wb_ref, hw_ref, wout_ref, bout_ref, out_ref):
    f32 = jnp.float32

    def qbc(lo, w):
        q = qa_ref[:, lo:lo + w]
        return jnp.broadcast_to(q[:, None, :], (NB, K, w)).reshape(R, w)

    q3 = qbc(0, 192)
    qp0 = qbc(256, NKP)
    qp1 = qbc(384, NKP)
    qp2 = qbc(512, NKP)

    ze = ze_ref[...]
    bz = jnp.dot(ze, wb_ref[...],
                 preferred_element_type=f32)

    prod = q3 * ge_ref[:, 0:192]
    seg192 = (lax.broadcasted_iota(jnp.int32, (192, H), 0) // C_HID
              == lax.broadcasted_iota(jnp.int32, (192, H), 1)).astype(f32)
    aqk = jnp.dot(prod, seg192,
                  preferred_element_type=f32)

    X = ge_ref[:, 384:528]
    Y = ge_ref[:, 528:672]
    Z = ge_ref[:, 672:816]

    def rcol(i):
        return ro_ref[:, i:i + 1]

    rot0 = rcol(0) * X + rcol(1) * Y + rcol(2) * Z
    rot1 = rcol(3) * X + rcol(4) * Y + rcol(5) * Z
    rot2 = rcol(6) * X + rcol(7) * Y + rcol(8) * Z
    d0 = dr_ref[:, 0:1]
    d1 = dr_ref[:, 1:2]
    d2 = dr_ref[:, 2:3]
    c0 = d1 * rot2 - d2 * rot1
    c1 = d2 * rot0 - d0 * rot2
    c2 = d0 * rot1 - d1 * rot0

    sq = ((qp0 - c0[:, :NKP]) ** 2
          + (qp1 - c1[:, :NKP]) ** 2
          + (qp2 - c2[:, :NKP]) ** 2)
    seg48 = (lax.broadcasted_iota(jnp.int32, (NKP, H), 0) // PQ
             == lax.broadcasted_iota(jnp.int32, (NKP, H), 1)).astype(f32)
    ptat = jnp.dot(sq, seg48,
                   preferred_element_type=f32)
    hw = jnp.log(1.0 + jnp.exp(hw_ref[0:1, :H])) * PT_SCALE

    logits = (aqk * QK_SCALE + bz * B_SCALE - 0.5 * (ptat * hw)
              + INF * (pm_ref[:, 0:1] - 1.0))

    l3 = logits.reshape(NB, K, H)
    m = jnp.max(l3, axis=1, keepdims=True)
    e3 = jnp.exp(l3 - m)
    ssum = jnp.sum(e3, axis=1, keepdims=True)
    a = (e3 / ssum).reshape(R, H)

    exp192 = (lax.broadcasted_iota(jnp.int32, (H, 192), 1) // C_HID
              == lax.broadcasted_iota(jnp.int32, (H, 192), 0)).astype(f32)
    a192 = jnp.dot(a, exp192, preferred_element_type=f32)
    exp96 = (lax.broadcasted_iota(jnp.int32, (H, NVP), 1) // PV
             == lax.broadcasted_iota(jnp.int32, (H, NVP), 0)).astype(f32)
    a96 = jnp.dot(a, exp96, preferred_element_type=f32)

    o = (a192 * ge_ref[:, 192:384]).reshape(NB, K, 192).sum(axis=1)
    op0 = (a96 * c0[:, NKP:]).reshape(NB, K, NVP).sum(axis=1)
    op1 = (a96 * c1[:, NKP:]).reshape(NB, K, NVP).sum(axis=1)
    op2 = (a96 * c2[:, NKP:]).reshape(NB, K, NVP).sum(axis=1)
    onorm = jnp.sqrt(op0 * op0 + op1 * op1 + op2 * op2 + EPS)

    acc = jnp.dot(o, wout_ref[0:192, :], preferred_element_type=f32)
    acc += jnp.dot(op0, wout_ref[192:288, :], preferred_element_type=f32)
    acc += jnp.dot(op1, wout_ref[288:384, :], preferred_element_type=f32)
    acc += jnp.dot(op2, wout_ref[384:480, :], preferred_element_type=f32)
    acc += jnp.dot(onorm, wout_ref[480:576, :], preferred_element_type=f32)

    for h in range(H):
        oph = (a[:, h:h + 1] * ze).reshape(NB, K, C_Z).sum(axis=1)
        acc += jnp.dot(oph, wout_ref[576 + h * C_Z:576 + (h + 1) * C_Z, :],
                       preferred_element_type=f32)

    out_ref[...] = acc + bout_ref[0:1, :]


def _attn(qa, ge, ze, ro, dr, pm, w_b, hw_pad, w_out, bout_pad, n_nodes):
    cod = H * (C_Z + C_HID + PV * 4)
    return pl.pallas_call(
        _attn_body,
        grid=(n_nodes // NB,),
        in_specs=[
            pl.BlockSpec((NB, D_QA), lambda i: (i, 0)),
            pl.BlockSpec((R, D_TABP), lambda i: (i, 0)),
            pl.BlockSpec((R, C_Z), lambda i: (i, 0)),
            pl.BlockSpec((R, 9), lambda i: (i, 0)),
            pl.BlockSpec((R, 3), lambda i: (i, 0)),
            pl.BlockSpec((R, 1), lambda i: (i, 0)),
            pl.BlockSpec((C_Z, H), lambda i: (0, 0)),
            pl.BlockSpec((8, 16), lambda i: (0, 0)),
            pl.BlockSpec((cod, C_N), lambda i: (0, 0)),
            pl.BlockSpec((8, C_N), lambda i: (0, 0)),
        ],
        out_specs=pl.BlockSpec((NB, C_N), lambda i: (i, 0)),
        out_shape=jax.ShapeDtypeStruct((n_nodes, C_N), jnp.float32),
        compiler_params=pltpu.CompilerParams(
            dimension_semantics=("arbitrary",)),
    )(qa, ge, ze, ro, dr, pm, w_b, hw_pad, w_out, bout_pad)


def kernel(s, z_e, r, direction, rel_ori, pair_mask, E_idx,
           w_q, w_kv, w_q_pts, w_kv_pts, w_b, head_weights, w_out, b_out):
    del r
    s2 = s.reshape(N, C_N)
    w_qa = jnp.zeros((C_N, D_QA), jnp.float32)
    w_qa = w_qa.at[:, 0:192].set(w_q)
    w_qa = w_qa.at[:, 256:304].set(w_q_pts[:, 0:48])
    w_qa = w_qa.at[:, 384:432].set(w_q_pts[:, 48:96])
    w_qa = w_qa.at[:, 512:560].set(w_q_pts[:, 96:144])
    w_tab = jnp.concatenate(
        [w_kv[:, _KV_PERM], w_kv_pts[:, _KV_PTS_PERM],
         jnp.zeros((C_N, D_TABP - D_TAB), jnp.float32)], axis=1)

    qa, table = _project(s2, w_qa, w_tab)

    idx1d = E_idx.astype(jnp.int32).reshape(N * K)
    ze = z_e.reshape(N * K, C_Z)
    ro = rel_ori.reshape(N * K, 9)
    dr = direction.reshape(N * K, 3)
    pm = pair_mask.reshape(N * K, 1)
    hw_pad = jnp.zeros((8, 16), jnp.float32).at[0, :H].set(head_weights)
    bout_pad = jnp.zeros((8, C_N), jnp.float32).at[0].set(b_out)

    ge = _sc_gather(table, idx1d, N * K)
    out = _attn(qa, ge, ze, ro, dr, pm, w_b, hw_pad, w_out, bout_pad, N)
    return out.reshape(1, N, C_N)
